# baseline (speedup 1.0000x reference)
import functools

import jax
import jax.numpy as jnp
from jax import lax
from jax.experimental import pallas as pl
from jax.experimental.pallas import tpu as pltpu
from jax.experimental.pallas import tpu_sc as plsc

_N = 10000
_E = 320000
_D = 128

_NC = 2
_NS = 16
_NW = _NC * _NS
_EPW = _E // _NW
_CH = 125
_NCHUNK = _EPW // _CH
_NBUF = 3
_ZCH = 80
_ZR = 40

_LANES = 16


@functools.cache
def _make_spmm_sc():
    mesh = plsc.VectorSubcoreMesh(core_axis_name="c", subcore_axis_name="s")

    @functools.partial(
        pl.kernel,
        mesh=mesh,
        out_type=jax.ShapeDtypeStruct((_NC, _N, _D), jnp.float32),
        scratch_types=(
            [
                pltpu.VMEM((_NBUF, _CH), jnp.int32),
                pltpu.VMEM((_NBUF, _CH), jnp.int32),
                pltpu.VMEM((_NBUF, _CH, _D), jnp.float32),
                pltpu.VMEM_SHARED((_N, _D), jnp.float32),
            ]
            + [pltpu.SemaphoreType.DMA] * (4 * _NBUF + 1)
        ),
    )
    def _spmm_sc(h_hbm, dst_hbm, src_hbm, z_hbm, out_hbm,
                 srci, dsti, rows_v, acc_sh, *sems):
        _spmm_body(h_hbm, dst_hbm, src_hbm, z_hbm, out_hbm,
                   srci, dsti, rows_v, acc_sh,
                   sems[:_NBUF], sems[_NBUF:2 * _NBUF],
                   sems[2 * _NBUF:3 * _NBUF], sems[3 * _NBUF:4 * _NBUF],
                   sems[4 * _NBUF])

    return _spmm_sc


def _spmm_body(h_hbm, dst_hbm, src_hbm, z_hbm, out_hbm,
               srci, dsti, rows_v, acc_sh,
               gsems, ssems, srcsems, dstsems, zsem):
    c = lax.axis_index("c")
    s = lax.axis_index("s")
    wid = c * _NS + s

    n_rchunk = _N // _ZCH
    rk_hi = (n_rchunk + _NS - 1) // _NS

    def _cond(pred, fn):
        if isinstance(pred, (bool, int)):
            if pred:
                fn()
        else:
            pl.when(pred)(fn)

    def _fetch_src(g, b, sync=False):
        cp = pltpu.sync_copy if sync else pltpu.async_copy
        cp(src_hbm.at[wid, g], srci.at[b],
           *(() if sync else (srcsems[b],)))

    def _fetch_dst(g, b):
        pltpu.async_copy(dst_hbm.at[wid, g], dsti.at[b], dstsems[b])

    def _issue_gather(b):
        pltpu.async_copy(h_hbm.at[srci.at[b]], rows_v.at[b], gsems[b])

    def _wait_gather(b):
        pltpu.make_async_copy(
            h_hbm.at[srci.at[b]], rows_v.at[b], gsems[b]).wait()

    def _issue_scatter(b):
        pltpu.async_copy(rows_v.at[b], acc_sh.at[dsti.at[b]], ssems[b],
                         add=True)

    def _wait_scatter(b):
        pltpu.make_async_copy(
            rows_v.at[b], acc_sh.at[dsti.at[b]], ssems[b]).wait()

    def _wait_src(b):
        pltpu.make_async_copy(
            src_hbm.at[wid, 0], srci.at[b], srcsems[b]).wait()

    def _wait_dst(b):
        pltpu.make_async_copy(
            dst_hbm.at[wid, 0], dsti.at[b], dstsems[b]).wait()

    _fetch_src(0, 0, sync=True)
    _fetch_src(1, 1, sync=True)
    _issue_gather(0)
    _issue_gather(1)
    for _g in range(2, _NBUF):
        _fetch_src(_g, _g)
    for _g in range(_NBUF - 1):
        _fetch_dst(_g, _g)

    n_zchunk = _N // _ZR
    zk_hi = (n_zchunk + _NS - 1) // _NS

    def _zacc(k, carry):
        cid = s + _NS * k

        @pl.when(cid < n_zchunk)
        def _():
            pltpu.async_copy(z_hbm, acc_sh.at[pl.ds(cid * _ZR, _ZR)], zsem)

        return carry

    def _zacc_drain(k, carry):
        cid = s + _NS * k

        @pl.when(cid < n_zchunk)
        def _():
            pltpu.make_async_copy(
                z_hbm, acc_sh.at[pl.ds(cid * _ZR, _ZR)], zsem).wait()

        return carry

    lax.fori_loop(0, zk_hi, _zacc, 0)
    lax.fori_loop(0, zk_hi, _zacc_drain, 0)
    plsc.subcore_barrier()

    def _step(g, b):
        b2 = (b + 2) % _NBUF
        b3 = (b + _NBUF - 1) % _NBUF
        _wait_gather(b)
        _cond(g >= 1 if isinstance(g, int) else True,
              lambda: _wait_scatter(b3))
        _cond(g + _NBUF - 1 < _NCHUNK,
              lambda: _fetch_dst(g + _NBUF - 1, b3))
        _cond(g + _NBUF < _NCHUNK, lambda: _fetch_src(g + _NBUF, b))

        def _g2():
            _wait_src(b2)
            _issue_gather(b2)

        _cond(g + 2 < _NCHUNK, _g2)
        _wait_dst(b)
        _issue_scatter(b)

    for t in range(_NBUF):
        _step(t, t % _NBUF)

    def _outer(o, carry):
        g0 = o * _NBUF + _NBUF
        for b in range(_NBUF):
            _step(g0 + b, b)
        return carry

    lax.fori_loop(0, (_NCHUNK - _NBUF) // _NBUF, _outer, 0)
    _TAIL0 = _NBUF + ((_NCHUNK - _NBUF) // _NBUF) * _NBUF
    for t in range(_TAIL0, _NCHUNK):
        _step(t, t % _NBUF)
    _wait_scatter((_NCHUNK - 1) % _NBUF)
    plsc.subcore_barrier()

    def _wb(k, carry):
        cid = s + _NS * k

        @pl.when(cid < n_rchunk)
        def _():
            pltpu.async_copy(acc_sh.at[pl.ds(cid * _ZCH, _ZCH)],
                             out_hbm.at[c, pl.ds(cid * _ZCH, _ZCH)], zsem)

        return carry

    def _wb_drain(k, carry):
        cid = s + _NS * k

        @pl.when(cid < n_rchunk)
        def _():
            pltpu.make_async_copy(
                acc_sh.at[pl.ds(cid * _ZCH, _ZCH)],
                out_hbm.at[c, pl.ds(cid * _ZCH, _ZCH)], zsem).wait()

        return carry

    lax.fori_loop(0, rk_hi, _wb, 0)
    lax.fori_loop(0, rk_hi, _wb_drain, 0)


_BM = 2000


def _mm_x_body(x_ref, w_ref, o_ref):
    o_ref[...] = jnp.dot(x_ref[...], w_ref[...],
                         preferred_element_type=jnp.float32)


def _mm_tanh_body(p_ref, w_ref, o_ref):
    h = jnp.tanh(p_ref[0] + p_ref[1])
    o_ref[...] = jnp.dot(h, w_ref[...], preferred_element_type=jnp.float32)


def _mm_add_body(p_ref, w_ref, o_ref):
    h = p_ref[0] + p_ref[1]
    o_ref[...] = jnp.dot(h, w_ref[...], preferred_element_type=jnp.float32)


def _softmax_body(p_ref, o_ref):
    h = p_ref[0] + p_ref[1]
    m = jnp.max(h, axis=1, keepdims=True)
    e = jnp.exp(h - m)
    o_ref[...] = e / jnp.sum(e, axis=1, keepdims=True)


_w_spec = pl.BlockSpec((_D, _D), lambda i: (0, 0))
_row_spec = pl.BlockSpec((_BM, _D), lambda i: (i, 0))
_pair_spec = pl.BlockSpec((_NC, _BM, _D), lambda i: (0, i, 0))
_grid = (_N // _BM,)
_out_nd = jax.ShapeDtypeStruct((_N, _D), jnp.float32)


def _mm_x(x, w):
    return pl.pallas_call(
        _mm_x_body, grid=_grid, out_shape=_out_nd,
        in_specs=[_row_spec, _w_spec], out_specs=_row_spec)(x, w)


def _mm_tanh(p, w):
    return pl.pallas_call(
        _mm_tanh_body, grid=_grid, out_shape=_out_nd,
        in_specs=[_pair_spec, _w_spec], out_specs=_row_spec)(p, w)


def _mm_add(p, w):
    return pl.pallas_call(
        _mm_add_body, grid=_grid, out_shape=_out_nd,
        in_specs=[_pair_spec, _w_spec], out_specs=_row_spec)(p, w)


def _softmax(p):
    return pl.pallas_call(
        _softmax_body, grid=_grid, out_shape=_out_nd,
        in_specs=[_pair_spec], out_specs=_row_spec)(p)


def kernel(x, edge_index, W1, W2, W3):
    ei = edge_index.astype(jnp.int32)
    dst3 = ei[0].reshape(_NW, _NCHUNK, _CH)
    src3 = ei[1].reshape(_NW, _NCHUNK, _CH)

    z = jnp.zeros((_ZR, _D), jnp.float32)
    spmm = _make_spmm_sc()
    h = _mm_x(x, W1)
    p = spmm(h, dst3, src3, z)
    h = _mm_tanh(p, W2)
    p = spmm(h, dst3, src3, z)
    h = _mm_add(p, W3)
    p = spmm(h, dst3, src3, z)
    return _softmax(p)


# speedup vs baseline: 1.3155x; 1.3155x over previous
import functools

import jax
import jax.numpy as jnp
from jax import lax
from jax.experimental import pallas as pl
from jax.experimental.pallas import tpu as pltpu
from jax.experimental.pallas import tpu_sc as plsc

_N = 10000
_E = 320000
_D = 128

_NC = 2
_NS = 16
_NW = _NC * _NS
_EPW = _E // _NW
_CH = 125
_NCHUNK = _EPW // _CH
_NBUF = 3
_ZCH = 80
_ZR = 40

_LANES = 16


@functools.cache
def _make_spmm_sc():
    mesh = plsc.VectorSubcoreMesh(core_axis_name="c", subcore_axis_name="s")

    @functools.partial(
        pl.kernel,
        mesh=mesh,
        out_type=jax.ShapeDtypeStruct((_NC, _N, _D), jnp.float32),
        scratch_types=(
            [
                pltpu.VMEM((_NBUF, _CH), jnp.int32),
                pltpu.VMEM((_NBUF, _CH), jnp.int32),
                pltpu.VMEM((_NBUF, _CH, _D), jnp.float32),
                pltpu.VMEM_SHARED((_N, _D), jnp.float32),
            ]
            + [pltpu.SemaphoreType.DMA] * (4 * _NBUF + 1)
        ),
    )
    def _spmm_sc(h_hbm, dst_hbm, src_hbm, out_hbm,
                 srci, dsti, rows_v, acc_sh, *sems):
        _spmm_body(h_hbm, dst_hbm, src_hbm, out_hbm,
                   srci, dsti, rows_v, acc_sh,
                   sems[:_NBUF], sems[_NBUF:2 * _NBUF],
                   sems[2 * _NBUF:3 * _NBUF], sems[3 * _NBUF:4 * _NBUF],
                   sems[4 * _NBUF])

    return _spmm_sc


def _spmm_body(h_hbm, dst_hbm, src_hbm, out_hbm,
               srci, dsti, rows_v, acc_sh,
               gsems, ssems, srcsems, dstsems, zsem):
    c = lax.axis_index("c")
    s = lax.axis_index("s")
    wid = c * _NS + s

    n_rchunk = _N // _ZCH
    rk_hi = (n_rchunk + _NS - 1) // _NS

    def _cond(pred, fn):
        if isinstance(pred, (bool, int)):
            if pred:
                fn()
        else:
            pl.when(pred)(fn)

    def _fetch_src(g, b, sync=False):
        cp = pltpu.sync_copy if sync else pltpu.async_copy
        cp(src_hbm.at[wid, g], srci.at[b],
           *(() if sync else (srcsems[b],)))

    def _fetch_dst(g, b):
        pltpu.async_copy(dst_hbm.at[wid, g], dsti.at[b], dstsems[b])

    def _issue_gather(b):
        pltpu.async_copy(h_hbm.at[srci.at[b]], rows_v.at[b], gsems[b])

    def _wait_gather(b):
        pltpu.make_async_copy(
            h_hbm.at[srci.at[b]], rows_v.at[b], gsems[b]).wait()

    def _issue_scatter(b):
        pltpu.async_copy(rows_v.at[b], acc_sh.at[dsti.at[b]], ssems[b],
                         add=True)

    def _wait_scatter(b):
        pltpu.make_async_copy(
            rows_v.at[b], acc_sh.at[dsti.at[b]], ssems[b]).wait()

    def _wait_src(b):
        pltpu.make_async_copy(
            src_hbm.at[wid, 0], srci.at[b], srcsems[b]).wait()

    def _wait_dst(b):
        pltpu.make_async_copy(
            dst_hbm.at[wid, 0], dsti.at[b], dstsems[b]).wait()

    _fetch_src(0, 0, sync=True)
    _fetch_src(1, 1, sync=True)
    _issue_gather(0)
    _issue_gather(1)
    for _g in range(2, _NBUF):
        _fetch_src(_g, _g)
    for _g in range(_NBUF - 1):
        _fetch_dst(_g, _g)

    zbuf = rows_v.at[_NBUF - 1, pl.ds(0, _ZR)]

    def _zrow(r, carry):
        for j in range(_D // _LANES):
            rows_v[_NBUF - 1, r, pl.ds(j * _LANES, _LANES)] = jnp.zeros(
                (_LANES,), jnp.float32)
        return carry

    lax.fori_loop(0, _ZR, _zrow, 0)

    n_zchunk = _N // _ZR
    zk_hi = (n_zchunk + _NS - 1) // _NS

    def _zacc(k, carry):
        cid = s + _NS * k

        @pl.when(cid < n_zchunk)
        def _():
            pltpu.async_copy(zbuf, acc_sh.at[pl.ds(cid * _ZR, _ZR)], zsem)

        return carry

    def _zacc_drain(k, carry):
        cid = s + _NS * k

        @pl.when(cid < n_zchunk)
        def _():
            pltpu.make_async_copy(
                zbuf, acc_sh.at[pl.ds(cid * _ZR, _ZR)], zsem).wait()

        return carry

    lax.fori_loop(0, zk_hi, _zacc, 0)
    lax.fori_loop(0, zk_hi, _zacc_drain, 0)
    plsc.subcore_barrier()

    def _step(g, b):
        b2 = (b + 2) % _NBUF
        b3 = (b + _NBUF - 1) % _NBUF
        _wait_gather(b)
        _cond(g >= 1 if isinstance(g, int) else True,
              lambda: _wait_scatter(b3))
        _cond(g + _NBUF - 1 < _NCHUNK,
              lambda: _fetch_dst(g + _NBUF - 1, b3))
        _cond(g + _NBUF < _NCHUNK, lambda: _fetch_src(g + _NBUF, b))

        def _g2():
            _wait_src(b2)
            _issue_gather(b2)

        _cond(g + 2 < _NCHUNK, _g2)
        _wait_dst(b)
        _issue_scatter(b)

    for t in range(_NBUF):
        _step(t, t % _NBUF)

    def _outer(o, carry):
        g0 = o * _NBUF + _NBUF
        for b in range(_NBUF):
            _step(g0 + b, b)
        return carry

    lax.fori_loop(0, (_NCHUNK - _NBUF) // _NBUF, _outer, 0)
    _TAIL0 = _NBUF + ((_NCHUNK - _NBUF) // _NBUF) * _NBUF
    for t in range(_TAIL0, _NCHUNK):
        _step(t, t % _NBUF)
    _wait_scatter((_NCHUNK - 1) % _NBUF)
    plsc.subcore_barrier()

    def _wb(k, carry):
        cid = s + _NS * k

        @pl.when(cid < n_rchunk)
        def _():
            pltpu.async_copy(acc_sh.at[pl.ds(cid * _ZCH, _ZCH)],
                             out_hbm.at[c, pl.ds(cid * _ZCH, _ZCH)], zsem)

        return carry

    def _wb_drain(k, carry):
        cid = s + _NS * k

        @pl.when(cid < n_rchunk)
        def _():
            pltpu.make_async_copy(
                acc_sh.at[pl.ds(cid * _ZCH, _ZCH)],
                out_hbm.at[c, pl.ds(cid * _ZCH, _ZCH)], zsem).wait()

        return carry

    lax.fori_loop(0, rk_hi, _wb, 0)
    lax.fori_loop(0, rk_hi, _wb_drain, 0)


_BM = 2000


def _mm_x_body(x_ref, w_ref, o_ref):
    o_ref[...] = jnp.dot(x_ref[...], w_ref[...],
                         preferred_element_type=jnp.float32)


def _mm_tanh_body(p_ref, w_ref, o_ref):
    h = jnp.tanh(p_ref[0] + p_ref[1])
    o_ref[...] = jnp.dot(h, w_ref[...], preferred_element_type=jnp.float32)


def _mm_add_body(p_ref, w_ref, o_ref):
    h = p_ref[0] + p_ref[1]
    o_ref[...] = jnp.dot(h, w_ref[...], preferred_element_type=jnp.float32)


def _softmax_body(p_ref, o_ref):
    h = p_ref[0] + p_ref[1]
    m = jnp.max(h, axis=1, keepdims=True)
    e = jnp.exp(h - m)
    o_ref[...] = e / jnp.sum(e, axis=1, keepdims=True)


_w_spec = pl.BlockSpec((_D, _D), lambda i: (0, 0))
_row_spec = pl.BlockSpec((_BM, _D), lambda i: (i, 0))
_pair_spec = pl.BlockSpec((_NC, _BM, _D), lambda i: (0, i, 0))
_grid = (_N // _BM,)
_out_nd = jax.ShapeDtypeStruct((_N, _D), jnp.float32)


def _mm_x(x, w):
    return pl.pallas_call(
        _mm_x_body, grid=_grid, out_shape=_out_nd,
        in_specs=[_row_spec, _w_spec], out_specs=_row_spec)(x, w)


def _mm_tanh(p, w):
    return pl.pallas_call(
        _mm_tanh_body, grid=_grid, out_shape=_out_nd,
        in_specs=[_pair_spec, _w_spec], out_specs=_row_spec)(p, w)


def _mm_add(p, w):
    return pl.pallas_call(
        _mm_add_body, grid=_grid, out_shape=_out_nd,
        in_specs=[_pair_spec, _w_spec], out_specs=_row_spec)(p, w)


def _softmax(p):
    return pl.pallas_call(
        _softmax_body, grid=_grid, out_shape=_out_nd,
        in_specs=[_pair_spec], out_specs=_row_spec)(p)


def kernel(x, edge_index, W1, W2, W3):
    ei = edge_index.astype(jnp.int32)
    dst3 = ei[0].reshape(_NW, _NCHUNK, _CH)
    src3 = ei[1].reshape(_NW, _NCHUNK, _CH)

    spmm = _make_spmm_sc()
    h = _mm_x(x, W1)
    p = spmm(h, dst3, src3)
    h = _mm_tanh(p, W2)
    p = spmm(h, dst3, src3)
    h = _mm_add(p, W3)
    p = spmm(h, dst3, src3)
    return _softmax(p)


# speedup vs baseline: 1.3516x; 1.0275x over previous
import functools

import jax
import jax.numpy as jnp
from jax import lax
from jax.experimental import pallas as pl
from jax.experimental.pallas import tpu as pltpu
from jax.experimental.pallas import tpu_sc as plsc

_N = 10000
_E = 320000
_D = 128

_NC = 2
_NS = 16
_NW = _NC * _NS
_EPW = _E // _NW
_CH = 125
_NCHUNK = _EPW // _CH
_NBUF = 3
_ZCH = 80
_ZR = 40

_LANES = 16


@functools.cache
def _make_spmm_sc():
    mesh = plsc.VectorSubcoreMesh(core_axis_name="c", subcore_axis_name="s")

    @functools.partial(
        pl.kernel,
        mesh=mesh,
        out_type=jax.ShapeDtypeStruct((_NC, _N, _D), jnp.float32),
        scratch_types=(
            [
                pltpu.VMEM((_NBUF, _CH), jnp.int32),
                pltpu.VMEM((_NBUF, _CH), jnp.int32),
                pltpu.VMEM((_NBUF, _CH, _D), jnp.float32),
                pltpu.VMEM_SHARED((_N, _D), jnp.float32),
            ]
            + [pltpu.SemaphoreType.DMA] * (4 * _NBUF + 1)
        ),
    )
    def _spmm_sc(h_hbm, dst_hbm, src_hbm, out_hbm,
                 srci, dsti, rows_v, acc_sh, *sems):
        _spmm_body(h_hbm, dst_hbm, src_hbm, out_hbm,
                   srci, dsti, rows_v, acc_sh,
                   sems[:_NBUF], sems[_NBUF:2 * _NBUF],
                   sems[2 * _NBUF:3 * _NBUF], sems[3 * _NBUF:4 * _NBUF],
                   sems[4 * _NBUF])

    return _spmm_sc


def _spmm_body(h_hbm, dst_hbm, src_hbm, out_hbm,
               srci, dsti, rows_v, acc_sh,
               gsems, ssems, srcsems, dstsems, zsem):
    c = lax.axis_index("c")
    s = lax.axis_index("s")
    wid = c * _NS + s

    n_rchunk = _N // _ZCH
    rk_hi = (n_rchunk + _NS - 1) // _NS

    def _cond(pred, fn):
        if isinstance(pred, (bool, int)):
            if pred:
                fn()
        else:
            pl.when(pred)(fn)

    def _fetch_src(g, b, sync=False):
        cp = pltpu.sync_copy if sync else pltpu.async_copy
        cp(src_hbm.at[wid, g], srci.at[b],
           *(() if sync else (srcsems[b],)))

    def _fetch_dst(g, b):
        pltpu.async_copy(dst_hbm.at[wid, g], dsti.at[b], dstsems[b])

    def _issue_gather(b):
        pltpu.async_copy(h_hbm.at[srci.at[b]], rows_v.at[b], gsems[b])

    def _wait_gather(b):
        pltpu.make_async_copy(
            h_hbm.at[srci.at[b]], rows_v.at[b], gsems[b]).wait()

    def _issue_scatter(b):
        pltpu.async_copy(rows_v.at[b], acc_sh.at[dsti.at[b]], ssems[b],
                         add=True)

    def _wait_scatter(b):
        pltpu.make_async_copy(
            rows_v.at[b], acc_sh.at[dsti.at[b]], ssems[b]).wait()

    def _wait_src(b):
        pltpu.make_async_copy(
            src_hbm.at[wid, 0], srci.at[b], srcsems[b]).wait()

    def _wait_dst(b):
        pltpu.make_async_copy(
            dst_hbm.at[wid, 0], dsti.at[b], dstsems[b]).wait()

    _fetch_src(0, 0, sync=True)
    _fetch_src(1, 1, sync=True)
    _issue_gather(0)
    _issue_gather(1)
    for _g in range(2, _NBUF):
        _fetch_src(_g, _g)
    for _g in range(_NBUF - 1):
        _fetch_dst(_g, _g)

    zbuf = rows_v.at[_NBUF - 1, pl.ds(0, _ZR)]

    def _zrow(r, carry):
        for j in range(_D // _LANES):
            rows_v[_NBUF - 1, r, pl.ds(j * _LANES, _LANES)] = jnp.zeros(
                (_LANES,), jnp.float32)
        return carry

    lax.fori_loop(0, _ZR, _zrow, 0)

    n_zchunk = _N // _ZR
    zk_hi = (n_zchunk + _NS - 1) // _NS

    def _zacc(k, carry):
        cid = s + _NS * k

        @pl.when(cid < n_zchunk)
        def _():
            pltpu.async_copy(zbuf, acc_sh.at[pl.ds(cid * _ZR, _ZR)], zsem)

        return carry

    def _zacc_drain(k, carry):
        cid = s + _NS * k

        @pl.when(cid < n_zchunk)
        def _():
            pltpu.make_async_copy(
                zbuf, acc_sh.at[pl.ds(cid * _ZR, _ZR)], zsem).wait()

        return carry

    lax.fori_loop(0, zk_hi, _zacc, 0)
    lax.fori_loop(0, zk_hi, _zacc_drain, 0)
    plsc.subcore_barrier()

    def _step(g, b):
        b2 = (b + 2) % _NBUF
        b3 = (b + _NBUF - 1) % _NBUF
        _wait_gather(b)
        _cond(g >= 1 if isinstance(g, int) else True,
              lambda: _wait_scatter(b3))
        _cond(g + _NBUF - 1 < _NCHUNK,
              lambda: _fetch_dst(g + _NBUF - 1, b3))
        _cond(g + _NBUF < _NCHUNK, lambda: _fetch_src(g + _NBUF, b))

        def _g2():
            _wait_src(b2)
            _issue_gather(b2)

        _cond(g + 2 < _NCHUNK, _g2)
        _wait_dst(b)
        _issue_scatter(b)

    for t in range(_NBUF):
        _step(t, t % _NBUF)

    def _outer(o, carry):
        g0 = o * _NBUF + _NBUF
        for b in range(_NBUF):
            _step(g0 + b, b)
        return carry

    lax.fori_loop(0, (_NCHUNK - _NBUF) // _NBUF, _outer, 0)
    _TAIL0 = _NBUF + ((_NCHUNK - _NBUF) // _NBUF) * _NBUF
    for t in range(_TAIL0, _NCHUNK):
        _step(t, t % _NBUF)
    _wait_scatter((_NCHUNK - 1) % _NBUF)
    plsc.subcore_barrier()

    def _wb(k, carry):
        cid = s + _NS * k

        @pl.when(cid < n_rchunk)
        def _():
            pltpu.async_copy(acc_sh.at[pl.ds(cid * _ZCH, _ZCH)],
                             out_hbm.at[c, pl.ds(cid * _ZCH, _ZCH)], zsem)

        return carry

    def _wb_drain(k, carry):
        cid = s + _NS * k

        @pl.when(cid < n_rchunk)
        def _():
            pltpu.make_async_copy(
                acc_sh.at[pl.ds(cid * _ZCH, _ZCH)],
                out_hbm.at[c, pl.ds(cid * _ZCH, _ZCH)], zsem).wait()

        return carry

    lax.fori_loop(0, rk_hi, _wb, 0)
    lax.fori_loop(0, rk_hi, _wb_drain, 0)


_BM = 5000


def _mm_x_body(x_ref, w_ref, o_ref):
    o_ref[...] = jnp.dot(x_ref[...], w_ref[...],
                         preferred_element_type=jnp.float32)


def _mm_tanh_body(p_ref, w_ref, o_ref):
    h = jnp.tanh(p_ref[0] + p_ref[1])
    o_ref[...] = jnp.dot(h, w_ref[...], preferred_element_type=jnp.float32)


def _mm_add_body(p_ref, w_ref, o_ref):
    h = p_ref[0] + p_ref[1]
    o_ref[...] = jnp.dot(h, w_ref[...], preferred_element_type=jnp.float32)


def _softmax_body(p_ref, o_ref):
    h = p_ref[0] + p_ref[1]
    m = jnp.max(h, axis=1, keepdims=True)
    e = jnp.exp(h - m)
    o_ref[...] = e / jnp.sum(e, axis=1, keepdims=True)


_w_spec = pl.BlockSpec((_D, _D), lambda i: (0, 0))
_row_spec = pl.BlockSpec((_BM, _D), lambda i: (i, 0))
_pair_spec = pl.BlockSpec((_NC, _BM, _D), lambda i: (0, i, 0))
_grid = (_N // _BM,)
_out_nd = jax.ShapeDtypeStruct((_N, _D), jnp.float32)


def _mm_x(x, w):
    return pl.pallas_call(
        _mm_x_body, grid=_grid, out_shape=_out_nd,
        in_specs=[_row_spec, _w_spec], out_specs=_row_spec)(x, w)


def _mm_tanh(p, w):
    return pl.pallas_call(
        _mm_tanh_body, grid=_grid, out_shape=_out_nd,
        in_specs=[_pair_spec, _w_spec], out_specs=_row_spec)(p, w)


def _mm_add(p, w):
    return pl.pallas_call(
        _mm_add_body, grid=_grid, out_shape=_out_nd,
        in_specs=[_pair_spec, _w_spec], out_specs=_row_spec)(p, w)


def _softmax(p):
    return pl.pallas_call(
        _softmax_body, grid=_grid, out_shape=_out_nd,
        in_specs=[_pair_spec], out_specs=_row_spec)(p)


def kernel(x, edge_index, W1, W2, W3):
    ei = edge_index.astype(jnp.int32)
    dst3 = ei[0].reshape(_NW, _NCHUNK, _CH)
    src3 = ei[1].reshape(_NW, _NCHUNK, _CH)

    spmm = _make_spmm_sc()
    h = _mm_x(x, W1)
    p = spmm(h, dst3, src3)
    h = _mm_tanh(p, W2)
    p = spmm(h, dst3, src3)
    h = _mm_add(p, W3)
    p = spmm(h, dst3, src3)
    return _softmax(p)
